# Initial kernel scaffold; baseline (speedup 1.0000x reference)
import jax
import jax.numpy as jnp
from jax.experimental import pallas as pl


def kernel(external_current, V_soma, V_ais, refrac_until, syn_g, syn_w, syn_pre_idx, syn_post_idx):
    raise NotImplementedError("write your pallas kernel here")


# speedup vs baseline: 308.1730x; 308.1730x over previous
import math

import jax
import jax.numpy as jnp
from jax import lax
from jax.experimental import pallas as pl
from jax.experimental.pallas import tpu as pltpu
from jax.experimental.pallas import tpu_sc as plsc

_N = 100000
_E = 3200000
_DT = 1e-4
_SLOPE = 4.0
_C_M_SOMA = 1.0
_G_L_SOMA = 0.05
_E_L_SOMA = -70.0
_C_M_AIS = 0.5
_G_L_AIS = 0.05
_E_L_AIS = -70.0
_V_T = -50.0
_DELTA_T = 2.0
_V_SPIKE = 20.0
_V_RESET = -60.0
_G_C = 0.1
_G_MAX = 0.01
_E_REV = 0.0
_TAU_DECAY = 0.005
_DECAY = math.exp(-_DT / _TAU_DECAY)

_NC = 2
_NS = 16
_NW = _NC * _NS
_EPW = _E // _NW
_CHUNK = 2000
_NCHUNK = _EPW // _CHUNK
_GROUPS = _CHUNK // 16
_UNROLL = 5
_ZUNROLL = 10

_ROWS = 8
_COLS = _N // _ROWS

def _mesh():
    return plsc.VectorSubcoreMesh(
        core_axis_name="c", subcore_axis_name="s",
        num_cores=_NC, num_subcores=_NS)


def _seg_body(g_hbm, w_hbm, idx_hbm, out_hbm, acc,
              gb0, gb1, wb0, wb1, ib0, ib1, sem0, sem1):
    wid = lax.axis_index("s") * _NC + lax.axis_index("c")
    base = wid * _EPW

    zeros16 = jnp.zeros((16,), jnp.float32)

    def zbody(k, carry):
        for u in range(_ZUNROLL):
            acc[pl.ds(k * (16 * _ZUNROLL) + u * 16, 16)] = zeros16
        return carry

    lax.fori_loop(0, _N // (16 * _ZUNROLL), zbody, 0)

    sems = (sem0, sem1)
    gbs = (gb0, gb1)
    wbs = (wb0, wb1)
    ibs = (ib0, ib1)

    def _chunk_copies(cidx, sl):
        off = pl.multiple_of(base + cidx * _CHUNK, _CHUNK)
        s = sems[sl]
        return (
            pltpu.make_async_copy(g_hbm.at[pl.ds(off, _CHUNK)], gbs[sl], s),
            pltpu.make_async_copy(w_hbm.at[pl.ds(off, _CHUNK)], wbs[sl], s),
            pltpu.make_async_copy(idx_hbm.at[pl.ds(off, _CHUNK)], ibs[sl], s),
        )

    def _start(cidx, sl):
        for cp in _chunk_copies(cidx, sl):
            cp.start()

    def _wait(cidx, sl):
        for cp in _chunk_copies(cidx, sl):
            cp.wait()

    _start(0, 0)

    def outer(i, carry):
        c0 = i * 2
        for sl in range(2):
            cidx = c0 + sl
            nxt = cidx + 1

            @pl.when(nxt < _NCHUNK)
            def _():
                _start(nxt, 1 - sl)

            _wait(cidx, sl)

            def grp(j, gcarry):
                for u in range(_UNROLL):
                    o = j * (16 * _UNROLL) + u * 16
                    iv = ibs[sl][pl.ds(o, 16)]
                    vals = gbs[sl][pl.ds(o, 16)] * wbs[sl][pl.ds(o, 16)]
                    plsc.addupdate_scatter(acc, [iv], vals)
                return gcarry

            lax.fori_loop(0, _GROUPS // _UNROLL, grp, 0)
        return carry

    lax.fori_loop(0, _NCHUNK // 2, outer, 0)
    pltpu.sync_copy(acc, out_hbm.at[wid])


def _segment_sum(syn_g, syn_w, syn_post_idx):
    f = pl.kernel(
        _seg_body,
        out_type=jax.ShapeDtypeStruct((_NW, _N), jnp.float32),
        mesh=_mesh(),
        compiler_params=pltpu.CompilerParams(needs_layout_passes=False),
        scratch_types=[
            pltpu.VMEM((_N,), jnp.float32),
            pltpu.VMEM((_CHUNK,), jnp.float32),
            pltpu.VMEM((_CHUNK,), jnp.float32),
            pltpu.VMEM((_CHUNK,), jnp.float32),
            pltpu.VMEM((_CHUNK,), jnp.float32),
            pltpu.VMEM((_CHUNK,), jnp.int32),
            pltpu.VMEM((_CHUNK,), jnp.int32),
            pltpu.SemaphoreType.DMA,
            pltpu.SemaphoreType.DMA,
        ],
    )
    return f(syn_g, syn_w, syn_post_idx)


def _dense_body(p_ref, ext_ref, vs_ref, va_ref, rf_ref,
                spk_ref, vso_ref, vao_ref, acc_ref):
    i = pl.program_id(0)

    @pl.when(i == 0)
    def _():
        acc_ref[...] = p_ref[0]

    @pl.when(i > 0)
    def _():
        acc_ref[...] = acc_ref[...] + p_ref[0]

    @pl.when(i == _NW - 1)
    def _():
        s = acc_ref[...]
        vs = vs_ref[...]
        va = va_ref[...]
        i_syn = (_G_MAX * _DECAY) * (_E_REV - vs) * s
        dvs = (_G_L_SOMA * (_E_L_SOMA - vs) + _G_C * (va - vs)
               + i_syn + ext_ref[...]) / _C_M_SOMA
        vs_new = vs + _DT * dvs
        ea = jnp.clip((va - _V_T) / _DELTA_T, -30.0, 30.0)
        i_exp = (_G_L_AIS * _DELTA_T) * jnp.exp(ea)
        dva = (_G_L_AIS * (_E_L_AIS - va) + i_exp
               + _G_C * (vs_new - va)) / _C_M_AIS
        va_new = va + _DT * dva
        va_new = jnp.where(rf_ref[...] > 0.0, _V_RESET, va_new)
        hard = (va_new >= _V_SPIKE).astype(jnp.float32)
        soft = 1.0 / (1.0 + jnp.exp(-_SLOPE * (va_new - _V_SPIKE)))
        spk_ref[...] = soft + (hard - soft)
        vso_ref[...] = vs_new
        vao_ref[...] = jnp.where(hard > 0.5, _V_RESET, va_new)


def _dense_update(partials, external_current, V_soma, V_ais, refrac_until):
    p3 = partials.reshape(_NW, _ROWS, _COLS)
    ext = external_current.reshape(_ROWS, _COLS)
    vs = V_soma.reshape(_ROWS, _COLS)
    va = V_ais.reshape(_ROWS, _COLS)
    rf = refrac_until.reshape(_ROWS, _COLS)
    dense_spec = pl.BlockSpec((_ROWS, _COLS), lambda i: (0, 0))
    out = pl.pallas_call(
        _dense_body,
        grid=(_NW,),
        in_specs=[
            pl.BlockSpec((1, _ROWS, _COLS), lambda i: (i, 0, 0)),
            dense_spec, dense_spec, dense_spec, dense_spec,
        ],
        out_specs=[dense_spec, dense_spec, dense_spec],
        out_shape=[jax.ShapeDtypeStruct((_ROWS, _COLS), jnp.float32)] * 3,
        scratch_shapes=[pltpu.VMEM((_ROWS, _COLS), jnp.float32)],
    )(p3, ext, vs, va, rf)
    return tuple(o.reshape(_N) for o in out)


def kernel(external_current, V_soma, V_ais, refrac_until,
           syn_g, syn_w, syn_pre_idx, syn_post_idx):
    partials = _segment_sum(syn_g, syn_w, syn_post_idx)
    spikes, vs_new, va_out = _dense_update(
        partials, external_current, V_soma, V_ais, refrac_until)
    return (spikes, vs_new, va_out)
